# baseline (speedup 1.0000x reference)
import jax
import jax.numpy as jnp
from jax.experimental import pallas as pl
from jax.experimental.pallas import tpu as pltpu

_NE = 1024
_D = 64
_R = 1024
_N = 16 * 1024
_CCOST = 0.25


def _vq_body(x_ref, w_ref, q_ref, idx_ref, md_ref, loss_ref, ppl_ref,
             w2_ref, t_ref, usage_ref, acc_ref):
    i = pl.program_id(0)

    @pl.when(i == 0)
    def _init():
        w = w_ref[...]
        w2_ref[...] = jnp.sum(w * w, axis=1)[None, :]
        rows = jax.lax.broadcasted_iota(jnp.int32, (_NE, 64), 0)
        lanes = jax.lax.broadcasted_iota(jnp.int32, (_NE, 64), 1)
        hi = (rows // 32).astype(jnp.float32)
        lo = (rows % 32).astype(jnp.float32)
        dec = jnp.where(lanes == 0, hi,
                        jnp.where(lanes == 1, lo,
                                  jnp.where(lanes == 2, 1.0, 0.0)))
        t_ref[...] = jnp.concatenate([w, dec], axis=1)
        usage_ref[...] = jnp.zeros_like(usage_ref)
        acc_ref[0] = 0.0
        acc_ref[1] = 0.0

    x = x_ref[...]
    w = w_ref[...]
    x2 = jnp.sum(x * x, axis=1, keepdims=True)
    dots = jax.lax.dot_general(-2.0 * x, w, (((1,), (1,)), ((), ())),
                               preferred_element_type=jnp.float32)
    dist = dots + w2_ref[...]
    mind0 = jnp.min(dist, axis=1, keepdims=True)
    mind = mind0 + x2
    validk = x2 > 1.0000001044244145e-12
    maskf = validk.astype(jnp.float32)
    ehm = jnp.where(dist == mind0, maskf, 0.0)
    s = jax.lax.dot_general(ehm, t_ref[...], (((1,), (0,)), ((), ())),
                            preferred_element_type=jnp.float32)
    aminf = s[:, 64:65] * 32.0 + s[:, 65:66]
    cnt = s[:, 66:67]
    q_ref[...] = s[:, 0:_D]
    idx_ref[...] = jnp.where(validk, aminf.astype(jnp.int32),
                             0)[:, 0][None, None, :]
    mdm = jnp.where(validk, mind, 0.0)
    md_ref[...] = mdm[:, 0][None, None, :]
    usage_ref[...] += jnp.sum(ehm.reshape(_R // 8, 8, _NE), axis=0)
    acc_ref[0] += jnp.sum(mdm)
    acc_ref[1] += jnp.sum(maskf)

    @pl.when(jnp.max(cnt) > 1.5)
    def _exact_ties():
        colsf = jax.lax.broadcasted_iota(jnp.int32,
                                         (1, _NE), 1).astype(jnp.float32)
        aminf2 = jnp.min(jnp.where(dist == mind0, colsf, float(_NE)),
                         axis=1, keepdims=True)
        aminf2_g = jnp.where(validk, aminf2, float(_NE))
        oh2 = (colsf == aminf2_g).astype(jnp.float32)
        qa2 = jax.lax.dot_general(oh2, w, (((1,), (0,)), ((), ())),
                                  preferred_element_type=jnp.float32)
        q_ref[...] = qa2
        idx_ref[...] = jnp.where(validk, aminf2.astype(jnp.int32),
                                 0)[:, 0][None, None, :]
        usage_ref[...] += jnp.sum((oh2 - ehm).reshape(_R // 8, 8, _NE),
                                  axis=0)

    @pl.when(i == pl.num_programs(0) - 1)
    def _fini():
        nv = jnp.maximum(acc_ref[1], 1.0)
        loss_ref[...] = jnp.full((1, 1), _CCOST / _D) * (acc_ref[0] / nv)
        avg = jnp.sum(usage_ref[...], axis=0)[None, :] / nv
        ent = -jnp.sum(avg * jnp.log(avg + 1e-10))
        ppl_ref[...] = jnp.exp(jnp.full((1, 1), 1.0) * ent)


_GRID = _N // _R

_vq_call = pl.pallas_call(
    _vq_body,
    grid=(_GRID,),
    in_specs=[pl.BlockSpec((_R, _D), lambda i: (i, 0)),
              pl.BlockSpec((_NE, _D), lambda i: (0, 0))],
    out_specs=[pl.BlockSpec((_R, _D), lambda i: (i, 0)),
               pl.BlockSpec((1, 1, _R), lambda i: (i, 0, 0)),
               pl.BlockSpec((1, 1, _R), lambda i: (i, 0, 0)),
               pl.BlockSpec((1, 1), lambda i: (0, 0)),
               pl.BlockSpec((1, 1), lambda i: (0, 0))],
    out_shape=[
        jax.ShapeDtypeStruct((_N, _D), jnp.float32),
        jax.ShapeDtypeStruct((_GRID, 1, _R), jnp.int32),
        jax.ShapeDtypeStruct((_GRID, 1, _R), jnp.float32),
        jax.ShapeDtypeStruct((1, 1), jnp.float32),
        jax.ShapeDtypeStruct((1, 1), jnp.float32),
    ],
    scratch_shapes=[pltpu.VMEM((1, _NE), jnp.float32),
                    pltpu.VMEM((_NE, 128), jnp.float32),
                    pltpu.VMEM((8, _NE), jnp.float32),
                    pltpu.SMEM((2,), jnp.float32)],
)


def kernel(inputs, W):
    shape = inputs.shape
    flat = inputs.reshape(-1, _D)
    q, idx, md, loss, ppl = _vq_call(flat, W)
    quantized = q.reshape(shape)
    indices = idx.reshape(shape[:-1])
    min_distances = md.reshape(shape[:-1])
    return (quantized, loss[0, 0], ppl[0, 0], indices, min_distances)


# speedup vs baseline: 1.1042x; 1.1042x over previous
import jax
import jax.numpy as jnp
from jax.experimental import pallas as pl
from jax.experimental.pallas import tpu as pltpu

_NE = 1024
_D = 64
_R = 1024
_N = 16 * 1024
_CCOST = 0.25


def _vq_body(x_ref, w_ref, q_ref, idx_ref, md_ref, loss_ref, ppl_ref,
             w2_ref, t_ref, usage_ref, acc_ref):
    i = pl.program_id(0)

    @pl.when(i == 0)
    def _init():
        w = w_ref[...]
        w2_ref[...] = jnp.sum(w * w, axis=1)[None, :]
        rows = jax.lax.broadcasted_iota(jnp.int32, (_NE, 128), 0)
        lanes = jax.lax.broadcasted_iota(jnp.int32, (_NE, 128), 1)
        hi = (rows // 32).astype(jnp.float32)
        lo = (rows % 32).astype(jnp.float32)
        t_ref[...] = jnp.where(lanes == 0, hi,
                               jnp.where(lanes == 1, lo,
                                         jnp.where(lanes == 2, 1.0, 0.0)))
        usage_ref[...] = jnp.zeros_like(usage_ref)
        acc_ref[0] = 0.0
        acc_ref[1] = 0.0

    x = x_ref[...]
    w = w_ref[...]
    x2 = jnp.sum(x * x, axis=1, keepdims=True)
    dots = jax.lax.dot_general(-2.0 * x, w, (((1,), (1,)), ((), ())),
                               preferred_element_type=jnp.float32)
    dist = dots + w2_ref[...]
    mind0 = jnp.min(dist, axis=1, keepdims=True)
    mind = mind0 + x2
    validk = x2 > 1.0000001044244145e-12
    maskf = validk.astype(jnp.float32)
    ehm = jnp.where(dist == mind0, maskf, 0.0)
    s = jax.lax.dot_general(ehm, t_ref[...], (((1,), (0,)), ((), ())),
                            preferred_element_type=jnp.float32)
    aminf = s[:, 0:1] * 32.0 + s[:, 1:2]
    cnt = s[:, 2:3]
    qa = jax.lax.dot_general(ehm, w, (((1,), (0,)), ((), ())),
                             preferred_element_type=jnp.float32)
    q_ref[...] = qa
    idx_ref[...] = jnp.where(validk, aminf.astype(jnp.int32),
                             0)[:, 0][None, None, :]
    mdm = jnp.where(validk, mind, 0.0)
    md_ref[...] = mdm[:, 0][None, None, :]
    usage_ref[...] += jnp.sum(ehm.reshape(_R // 8, 8, _NE), axis=0)
    acc_ref[0] += jnp.sum(mdm)
    acc_ref[1] += jnp.sum(maskf)

    @pl.when(jnp.max(cnt) > 1.5)
    def _exact_ties():
        colsf = jax.lax.broadcasted_iota(jnp.int32,
                                         (1, _NE), 1).astype(jnp.float32)
        aminf2 = jnp.min(jnp.where(dist == mind0, colsf, float(_NE)),
                         axis=1, keepdims=True)
        aminf2_g = jnp.where(validk, aminf2, float(_NE))
        oh2 = (colsf == aminf2_g).astype(jnp.float32)
        qa2 = jax.lax.dot_general(oh2, w, (((1,), (0,)), ((), ())),
                                  preferred_element_type=jnp.float32)
        q_ref[...] = qa2
        idx_ref[...] = jnp.where(validk, aminf2.astype(jnp.int32),
                                 0)[:, 0][None, None, :]
        usage_ref[...] += jnp.sum((oh2 - ehm).reshape(_R // 8, 8, _NE),
                                  axis=0)

    @pl.when(i == pl.num_programs(0) - 1)
    def _fini():
        nv = jnp.maximum(acc_ref[1], 1.0)
        loss_ref[...] = jnp.full((1, 1), _CCOST / _D) * (acc_ref[0] / nv)
        avg = jnp.sum(usage_ref[...], axis=0)[None, :] / nv
        ent = -jnp.sum(avg * jnp.log(avg + 1e-10))
        ppl_ref[...] = jnp.exp(jnp.full((1, 1), 1.0) * ent)


_GRID = _N // _R

_vq_call = pl.pallas_call(
    _vq_body,
    grid=(_GRID,),
    in_specs=[pl.BlockSpec((_R, _D), lambda i: (i, 0)),
              pl.BlockSpec((_NE, _D), lambda i: (0, 0))],
    out_specs=[pl.BlockSpec((_R, _D), lambda i: (i, 0)),
               pl.BlockSpec((1, 1, _R), lambda i: (i, 0, 0)),
               pl.BlockSpec((1, 1, _R), lambda i: (i, 0, 0)),
               pl.BlockSpec((1, 1), lambda i: (0, 0)),
               pl.BlockSpec((1, 1), lambda i: (0, 0))],
    out_shape=[
        jax.ShapeDtypeStruct((_N, _D), jnp.float32),
        jax.ShapeDtypeStruct((_GRID, 1, _R), jnp.int32),
        jax.ShapeDtypeStruct((_GRID, 1, _R), jnp.float32),
        jax.ShapeDtypeStruct((1, 1), jnp.float32),
        jax.ShapeDtypeStruct((1, 1), jnp.float32),
    ],
    scratch_shapes=[pltpu.VMEM((1, _NE), jnp.float32),
                    pltpu.VMEM((_NE, 128), jnp.float32),
                    pltpu.VMEM((8, _NE), jnp.float32),
                    pltpu.SMEM((2,), jnp.float32)],
)


def kernel(inputs, W):
    shape = inputs.shape
    flat = inputs.reshape(-1, _D)
    q, idx, md, loss, ppl = _vq_call(flat, W)
    quantized = q.reshape(shape)
    indices = idx.reshape(shape[:-1])
    min_distances = md.reshape(shape[:-1])
    return (quantized, loss[0, 0], ppl[0, 0], indices, min_distances)
